# baseline (speedup 1.0000x reference)
import functools

import jax
import jax.numpy as jnp
from jax import lax
from jax.experimental import pallas as pl
from jax.experimental.pallas import tpu as pltpu
from jax.experimental.pallas import tpu_sc as plsc

B = 128
V = 100000
K = 100
EPS = 1e-8

NC = 2
NS = 16
NW = NC * NS
RPW = B // NW
WIN = 10000
NWIN = V // WIN
GRP = 400
NGRP = WIN // GRP
NVR = GRP // 16
CAP = 2048
GPAD = 128
L = 16


def _sc_body(s_hbm, t_hbm, out_hbm, twina, swina, twinb, swinb, cval,
             csv, pbuf, gval, gsv, res, semta, semsa, semtb, semsb):
    wid = lax.axis_index("s") * NC + lax.axis_index("c")
    lane = lax.iota(jnp.int32, 16)

    def select_thr(n):
        def cp(i, c):
            pbuf[pl.ds(i * 16, 16)] = cval[pl.ds(i * 16, 16)]
            return c

        lax.fori_loop(0, CAP // 16, cp, 0)

        def w_cond(c):
            return jnp.logical_not(c[0])

        def w_body(c):
            _, an, r, thr, m = c
            v0 = pbuf[pl.ds(0, 16)]
            pivot = v0[0]
            nv = (an + 15) // 16

            def cnt(i, cc):
                cgtv, ceqv = cc
                x = pbuf[pl.ds(i * 16, 16)]
                valid = (lane + i * 16) < an
                gt = jnp.logical_and(valid, x > pivot)
                eq = jnp.logical_and(valid, x == pivot)
                return (cgtv + plsc.all_reduce_population_count(gt),
                        ceqv + plsc.all_reduce_population_count(eq))

            zi = jnp.zeros((16,), jnp.int32)
            cgtv, ceqv = lax.fori_loop(0, nv, cnt, (zi, zi))
            cgt = cgtv[0]
            ceq = ceqv[0]
            done = jnp.logical_and(cgt <= r, cgt + ceq > r)

            def finish(an, r, thr, m):
                return (jnp.bool_(True), an, r, pivot, r - cgt + 1)

            def recurse(an, r, thr, m):
                up = cgt > r

                def cmp(i, kpv):
                    x = pbuf[pl.ds(i * 16, 16)]
                    valid = (lane + i * 16) < an
                    gt = jnp.logical_and(valid, x > pivot)
                    lt = jnp.logical_and(valid, x < pivot)
                    keep = jnp.logical_or(
                        jnp.logical_and(gt, up),
                        jnp.logical_and(lt, jnp.logical_not(up)))
                    cum = plsc.cumsum(keep.astype(jnp.int32))
                    pos = kpv + cum - 1
                    plsc.store_scatter(pbuf, [pos], x, mask=keep)
                    return kpv + plsc.all_reduce_population_count(keep)

                kpv = lax.fori_loop(0, nv, cmp, jnp.zeros((16,), jnp.int32))
                kp = kpv[0]
                r2 = jnp.where(up, r, r - (cgt + ceq))
                return (jnp.bool_(False), kp, r2, thr, m)

            return lax.cond(done, finish, recurse, an, r, thr, m)

        carry = (jnp.bool_(False), n, jnp.int32(K - 1), jnp.float32(0.0),
                 jnp.int32(0))
        _, _, _, thr, m = lax.while_loop(w_cond, w_body, carry)
        return thr, m

    def compact(n, thr, m, dval, dsv):
        nv = (n + 15) // 16

        def body(i, cc):
            kpv, eqtv = cc
            x = cval[pl.ds(i * 16, 16)]
            sx = csv[pl.ds(i * 16, 16)]
            valid = (lane + i * 16) < n
            gt = jnp.logical_and(valid, x > thr)
            eq = jnp.logical_and(valid, x == thr)
            eqc = plsc.cumsum(eq.astype(jnp.int32)) + eqtv
            take = jnp.logical_and(eq, eqc <= m)
            keep = jnp.logical_or(gt, take)
            cum = plsc.cumsum(keep.astype(jnp.int32))
            pos = kpv + cum - 1
            plsc.store_scatter(dval, [pos], x, mask=keep)
            plsc.store_scatter(dsv, [pos], sx, mask=keep)
            return (kpv + plsc.all_reduce_population_count(keep),
                    eqtv + plsc.all_reduce_population_count(eq))

        zi = jnp.zeros((16,), jnp.int32)
        lax.fori_loop(0, nv, body, (zi, zi))

    def row_body(j, _):
        r = wid * RPW + j
        base = r * V

        pltpu.async_copy(t_hbm.at[pl.ds(base, WIN)], twina, semta)
        pltpu.async_copy(s_hbm.at[pl.ds(base, WIN)], swina, semsa)
        pltpu.async_copy(t_hbm.at[pl.ds(base + WIN, WIN)], twinb, semtb)
        pltpu.async_copy(s_hbm.at[pl.ds(base + WIN, WIN)], swinb, semsb)

        def do_window(w, ptr, thr, twin, swin, semt, sems):
            pltpu.make_async_copy(t_hbm.at[pl.ds(base + w * WIN, WIN)],
                                  twin, semt).wait()
            pltpu.make_async_copy(s_hbm.at[pl.ds(base + w * WIN, WIN)],
                                  swin, sems).wait()

            def grp(g, carry):
                ptr, thr = carry
                off = g * GRP
                xs = [twin[pl.ds(off + q * 16, 16)] for q in range(NVR)]
                ms = [x > thr for x in xs]
                while len(ms) > 1:
                    ms = [jnp.logical_or(a, b)
                          for a, b in zip(ms[::2], ms[1::2])] + (
                              [ms[-1]] if len(ms) % 2 else [])
                hit = plsc.all_reduce_population_count(ms[0])[0] > 0

                def hit_path(ptr0, thr0):
                    def rebuild(p0, t0):
                        t2, m2 = select_thr(p0)
                        compact(p0, t2, m2, cval, csv)
                        return jnp.int32(K), t2

                    def keep_pt(p0, t0):
                        return p0, t0

                    p1, t1 = lax.cond(ptr0 + GRP > CAP, rebuild, keep_pt,
                                      ptr0, thr0)
                    run = jnp.zeros((16,), jnp.int32) + p1
                    for q in range(NVR):
                        x = xs[q]
                        mm = x > t1
                        cum = plsc.cumsum(mm.astype(jnp.int32))
                        pos = run + cum - 1
                        plsc.store_scatter(cval, [pos], x, mask=mm)
                        sx = swin[pl.ds(off + q * 16, 16)]
                        plsc.store_scatter(csv, [pos], sx, mask=mm)
                        run = run + plsc.all_reduce_population_count(mm)
                    return run[0], t1

                def miss_path(ptr0, thr0):
                    return ptr0, thr0

                ptr, thr = lax.cond(hit, hit_path, miss_path, ptr, thr)
                return ptr, thr

            return lax.fori_loop(0, NGRP, grp, (ptr, thr))

        def win2_body(w2, carry):
            ptr, thr = carry
            w = w2 * 2
            ptr, thr = do_window(w, ptr, thr, twina, swina, semta, semsa)

            @pl.when(w + 2 < NWIN)
            def _():
                pltpu.async_copy(t_hbm.at[pl.ds(base + (w + 2) * WIN, WIN)],
                                 twina, semta)
                pltpu.async_copy(s_hbm.at[pl.ds(base + (w + 2) * WIN, WIN)],
                                 swina, semsa)

            ptr, thr = do_window(w + 1, ptr, thr, twinb, swinb, semtb,
                                 semsb)

            @pl.when(w + 3 < NWIN)
            def _():
                pltpu.async_copy(t_hbm.at[pl.ds(base + (w + 3) * WIN, WIN)],
                                 twinb, semtb)
                pltpu.async_copy(s_hbm.at[pl.ds(base + (w + 3) * WIN, WIN)],
                                 swinb, semsb)

            return ptr, thr

        ptr, thr = lax.fori_loop(0, NWIN // 2, win2_body,
                                 (jnp.int32(0), jnp.float32(-jnp.inf)))

        for q in range(GPAD // 16):
            gval[pl.ds(q * 16, 16)] = jnp.zeros((16,), jnp.float32)
            gsv[pl.ds(q * 16, 16)] = jnp.zeros((16,), jnp.float32)

        t2, m2 = select_thr(ptr)
        compact(ptr, t2, m2, gval, gsv)

        def acc(q, cc):
            d, ss, tt = cc
            tv = gval[pl.ds(q * 16, 16)]
            sv = gsv[pl.ds(q * 16, 16)]
            return d + sv * tv, ss + sv * sv, tt + tv * tv

        z = jnp.zeros((16,), jnp.float32)
        dv, ssv, ttv = lax.fori_loop(0, 7, acc, (z, z, z))
        dot = jnp.sum(dv)
        ssum = jnp.sum(ssv)
        tsum = jnp.sum(ttv)
        res[...] = jnp.where(
            lane == 0, dot,
            jnp.where(lane == 1, ssum, jnp.where(lane == 2, tsum, 0.0)))
        pltpu.sync_copy(res, out_hbm.at[pl.ds(r * L, L)])
        return 0

    lax.fori_loop(0, RPW, row_body, 0)


@functools.partial(
    pl.kernel,
    out_type=jax.ShapeDtypeStruct((B * L,), jnp.float32),
    mesh=plsc.VectorSubcoreMesh(core_axis_name="c", subcore_axis_name="s"),
    compiler_params=pltpu.CompilerParams(needs_layout_passes=False),
    scratch_types=[
        pltpu.VMEM((WIN,), jnp.float32),
        pltpu.VMEM((WIN,), jnp.float32),
        pltpu.VMEM((WIN,), jnp.float32),
        pltpu.VMEM((WIN,), jnp.float32),
        pltpu.VMEM((CAP + 16,), jnp.float32),
        pltpu.VMEM((CAP + 16,), jnp.float32),
        pltpu.VMEM((CAP + 16,), jnp.float32),
        pltpu.VMEM((GPAD,), jnp.float32),
        pltpu.VMEM((GPAD,), jnp.float32),
        pltpu.VMEM((L,), jnp.float32),
        pltpu.SemaphoreType.DMA,
        pltpu.SemaphoreType.DMA,
        pltpu.SemaphoreType.DMA,
        pltpu.SemaphoreType.DMA,
    ],
)
def _sc_topk(s_hbm, t_hbm, out_hbm, twina, swina, twinb, swinb, cval,
             csv, pbuf, gval, gsv, res, semta, semsa, semtb, semsb):
    _sc_body(s_hbm, t_hbm, out_hbm, twina, swina, twinb, swinb, cval,
             csv, pbuf, gval, gsv, res, semta, semsa, semtb, semsb)


def _cos_body(x_ref, o_ref):
    x = x_ref[...]
    dot = x[:, 0:1]
    ss = x[:, 1:2]
    tt = x[:, 2:3]
    nx = jnp.maximum(jnp.sqrt(ss), EPS)
    ny = jnp.maximum(jnp.sqrt(tt), EPS)
    cos = dot / (nx * ny)
    o_ref[...] = (jnp.sum(1.0 - cos) / B).reshape(1, 1)


def kernel(s_logits, t_logits):
    stats = _sc_topk(s_logits.reshape(-1), t_logits.reshape(-1))
    out = pl.pallas_call(
        _cos_body,
        out_shape=jax.ShapeDtypeStruct((1, 1), jnp.float32),
    )(stats.reshape(B, L))
    return out[0, 0]


# speedup vs baseline: 1.1399x; 1.1399x over previous
import functools

import jax
import jax.numpy as jnp
from jax import lax
from jax.experimental import pallas as pl
from jax.experimental.pallas import tpu as pltpu
from jax.experimental.pallas import tpu_sc as plsc

B = 128
V = 100000
K = 100
EPS = 1e-8

NC = 2
NS = 16
NW = NC * NS
RPW = B // NW
WIN = 10000
NWIN = V // WIN
GRP = 400
NGRP = WIN // GRP
NVR = GRP // 16
CAP = 2048
GPAD = 128
L = 16


def _sc_body(s_hbm, t_hbm, out_hbm, twina, twinb, cval, cidx, pbuf,
             gval, gidx, sdst, res, semta, semtb, semg):
    wid = lax.axis_index("s") * NC + lax.axis_index("c")
    lane = lax.iota(jnp.int32, 16)

    def select_thr(n):
        def cp(i, c):
            pbuf[pl.ds(i * 16, 16)] = cval[pl.ds(i * 16, 16)]
            return c

        lax.fori_loop(0, CAP // 16, cp, 0)

        def w_cond(c):
            return jnp.logical_not(c[0])

        def w_body(c):
            _, an, r, thr, m = c
            v0 = pbuf[pl.ds(0, 16)]
            pivot = v0[0]
            nv = (an + 15) // 16

            def cnt(i, cc):
                cgtv, ceqv = cc
                x = pbuf[pl.ds(i * 16, 16)]
                valid = (lane + i * 16) < an
                gt = jnp.logical_and(valid, x > pivot)
                eq = jnp.logical_and(valid, x == pivot)
                return (cgtv + plsc.all_reduce_population_count(gt),
                        ceqv + plsc.all_reduce_population_count(eq))

            zi = jnp.zeros((16,), jnp.int32)
            cgtv, ceqv = lax.fori_loop(0, nv, cnt, (zi, zi))
            cgt = cgtv[0]
            ceq = ceqv[0]
            done = jnp.logical_and(cgt <= r, cgt + ceq > r)

            def finish(an, r, thr, m):
                return (jnp.bool_(True), an, r, pivot, r - cgt + 1)

            def recurse(an, r, thr, m):
                up = cgt > r

                def cmp(i, kpv):
                    x = pbuf[pl.ds(i * 16, 16)]
                    valid = (lane + i * 16) < an
                    gt = jnp.logical_and(valid, x > pivot)
                    lt = jnp.logical_and(valid, x < pivot)
                    keep = jnp.logical_or(
                        jnp.logical_and(gt, up),
                        jnp.logical_and(lt, jnp.logical_not(up)))
                    cum = plsc.cumsum(keep.astype(jnp.int32))
                    pos = kpv + cum - 1
                    plsc.store_scatter(pbuf, [pos], x, mask=keep)
                    return kpv + plsc.all_reduce_population_count(keep)

                kpv = lax.fori_loop(0, nv, cmp, jnp.zeros((16,), jnp.int32))
                kp = kpv[0]
                r2 = jnp.where(up, r, r - (cgt + ceq))
                return (jnp.bool_(False), kp, r2, thr, m)

            return lax.cond(done, finish, recurse, an, r, thr, m)

        carry = (jnp.bool_(False), n, jnp.int32(K - 1), jnp.float32(0.0),
                 jnp.int32(0))
        _, _, _, thr, m = lax.while_loop(w_cond, w_body, carry)
        return thr, m

    def compact(n, thr, m, dval, didx):
        nv = (n + 15) // 16

        def body(i, cc):
            kpv, eqtv = cc
            x = cval[pl.ds(i * 16, 16)]
            sx = cidx[pl.ds(i * 16, 16)]
            valid = (lane + i * 16) < n
            gt = jnp.logical_and(valid, x > thr)
            eq = jnp.logical_and(valid, x == thr)
            eqc = plsc.cumsum(eq.astype(jnp.int32)) + eqtv
            take = jnp.logical_and(eq, eqc <= m)
            keep = jnp.logical_or(gt, take)
            cum = plsc.cumsum(keep.astype(jnp.int32))
            pos = kpv + cum - 1
            plsc.store_scatter(dval, [pos], x, mask=keep)
            plsc.store_scatter(didx, [pos], sx, mask=keep)
            return (kpv + plsc.all_reduce_population_count(keep),
                    eqtv + plsc.all_reduce_population_count(eq))

        zi = jnp.zeros((16,), jnp.int32)
        lax.fori_loop(0, nv, body, (zi, zi))

    def row_body(j, _):
        r = wid * RPW + j
        base = r * V

        pltpu.async_copy(t_hbm.at[pl.ds(base, WIN)], twina, semta)
        pltpu.async_copy(t_hbm.at[pl.ds(base + WIN, WIN)], twinb, semtb)

        def do_window(w, ptr, thr, twin, semt):
            pltpu.make_async_copy(t_hbm.at[pl.ds(base + w * WIN, WIN)],
                                  twin, semt).wait()

            def grp(g, carry):
                ptr, thr = carry
                off = g * GRP
                xs = [twin[pl.ds(off + q * 16, 16)] for q in range(NVR)]
                ms = [x > thr for x in xs]
                while len(ms) > 1:
                    ms = [jnp.logical_or(a, b)
                          for a, b in zip(ms[::2], ms[1::2])] + (
                              [ms[-1]] if len(ms) % 2 else [])
                hit = plsc.all_reduce_population_count(ms[0])[0] > 0

                def hit_path(ptr0, thr0):
                    def rebuild(p0, t0):
                        t2, m2 = select_thr(p0)
                        compact(p0, t2, m2, cval, cidx)
                        return jnp.int32(K), t2

                    def keep_pt(p0, t0):
                        return p0, t0

                    p1, t1 = lax.cond(ptr0 + GRP > CAP, rebuild, keep_pt,
                                      ptr0, thr0)
                    run = jnp.zeros((16,), jnp.int32) + p1
                    for q in range(NVR):
                        x = xs[q]
                        mm = x > t1
                        cum = plsc.cumsum(mm.astype(jnp.int32))
                        pos = run + cum - 1
                        plsc.store_scatter(cval, [pos], x, mask=mm)
                        colv = base + w * WIN + off + q * 16 + lane
                        plsc.store_scatter(cidx, [pos], colv, mask=mm)
                        run = run + plsc.all_reduce_population_count(mm)
                    return run[0], t1

                def miss_path(ptr0, thr0):
                    return ptr0, thr0

                ptr, thr = lax.cond(hit, hit_path, miss_path, ptr, thr)
                return ptr, thr

            return lax.fori_loop(0, NGRP, grp, (ptr, thr))

        def win2_body(w2, carry):
            ptr, thr = carry
            w = w2 * 2
            ptr, thr = do_window(w, ptr, thr, twina, semta)

            @pl.when(w + 2 < NWIN)
            def _():
                pltpu.async_copy(t_hbm.at[pl.ds(base + (w + 2) * WIN, WIN)],
                                 twina, semta)

            ptr, thr = do_window(w + 1, ptr, thr, twinb, semtb)

            @pl.when(w + 3 < NWIN)
            def _():
                pltpu.async_copy(t_hbm.at[pl.ds(base + (w + 3) * WIN, WIN)],
                                 twinb, semtb)

            return ptr, thr

        ptr, thr = lax.fori_loop(0, NWIN // 2, win2_body,
                                 (jnp.int32(0), jnp.float32(-jnp.inf)))

        for q in range(GPAD // 16):
            gval[pl.ds(q * 16, 16)] = jnp.zeros((16,), jnp.float32)
            gidx[pl.ds(q * 16, 16)] = jnp.zeros((16,), jnp.int32) + base

        t2, m2 = select_thr(ptr)
        compact(ptr, t2, m2, gval, gidx)

        pltpu.async_copy(s_hbm.at[gidx], sdst, semg).wait()

        def acc(q, cc):
            d, ss, tt = cc
            tv = gval[pl.ds(q * 16, 16)]
            sv = sdst[pl.ds(q * 16, 16)]
            valid = (lane + q * 16) < K
            sv = jnp.where(valid, sv, 0.0)
            return d + sv * tv, ss + sv * sv, tt + tv * tv

        z = jnp.zeros((16,), jnp.float32)
        dv, ssv, ttv = lax.fori_loop(0, 7, acc, (z, z, z))
        dot = jnp.sum(dv)
        ssum = jnp.sum(ssv)
        tsum = jnp.sum(ttv)
        res[...] = jnp.where(
            lane == 0, dot,
            jnp.where(lane == 1, ssum, jnp.where(lane == 2, tsum, 0.0)))
        pltpu.sync_copy(res, out_hbm.at[pl.ds(r * L, L)])
        return 0

    lax.fori_loop(0, RPW, row_body, 0)


@functools.partial(
    pl.kernel,
    out_type=jax.ShapeDtypeStruct((B * L,), jnp.float32),
    mesh=plsc.VectorSubcoreMesh(core_axis_name="c", subcore_axis_name="s"),
    compiler_params=pltpu.CompilerParams(needs_layout_passes=False),
    scratch_types=[
        pltpu.VMEM((WIN,), jnp.float32),
        pltpu.VMEM((WIN,), jnp.float32),
        pltpu.VMEM((CAP + 16,), jnp.float32),
        pltpu.VMEM((CAP + 16,), jnp.int32),
        pltpu.VMEM((CAP + 16,), jnp.float32),
        pltpu.VMEM((GPAD,), jnp.float32),
        pltpu.VMEM((GPAD,), jnp.int32),
        pltpu.VMEM((GPAD,), jnp.float32),
        pltpu.VMEM((L,), jnp.float32),
        pltpu.SemaphoreType.DMA,
        pltpu.SemaphoreType.DMA,
        pltpu.SemaphoreType.DMA,
    ],
)
def _sc_topk(s_hbm, t_hbm, out_hbm, twina, twinb, cval, cidx, pbuf,
             gval, gidx, sdst, res, semta, semtb, semg):
    _sc_body(s_hbm, t_hbm, out_hbm, twina, twinb, cval, cidx, pbuf,
             gval, gidx, sdst, res, semta, semtb, semg)


def _cos_body(x_ref, o_ref):
    x = x_ref[...]
    dot = x[:, 0:1]
    ss = x[:, 1:2]
    tt = x[:, 2:3]
    nx = jnp.maximum(jnp.sqrt(ss), EPS)
    ny = jnp.maximum(jnp.sqrt(tt), EPS)
    cos = dot / (nx * ny)
    o_ref[...] = (jnp.sum(1.0 - cos) / B).reshape(1, 1)


def kernel(s_logits, t_logits):
    stats = _sc_topk(s_logits.reshape(-1), t_logits.reshape(-1))
    out = pl.pallas_call(
        _cos_body,
        out_shape=jax.ShapeDtypeStruct((1, 1), jnp.float32),
    )(stats.reshape(B, L))
    return out[0, 0]
